# baseline (speedup 1.0000x reference)
import functools

import jax
import jax.numpy as jnp
from jax import lax
from jax.experimental import pallas as pl
from jax.experimental.pallas import tpu as pltpu

_NCHUNK = 16


def _body(S, U, BH, pos_ref, cache_any, upd_any, out_any, csem, wsem):
    bhc = BH // _NCHUNK
    copies = []
    for k in range(_NCHUNK):
        cp = pltpu.make_async_copy(
            cache_any.at[pl.ds(k * bhc, bhc)],
            out_any.at[pl.ds(k * bhc, bhc)],
            csem,
        )
        cp.start()
        copies.append(cp)
    for cp in copies:
        cp.wait()
    p = pos_ref[0]
    writes = []
    for i in range(U):
        r = lax.rem(p + i, S)
        wr = pltpu.make_async_copy(
            upd_any.at[:, pl.ds(i, 1), :],
            out_any.at[:, pl.ds(r, 1), :],
            wsem,
        )
        wr.start()
        writes.append(wr)
    for wr in writes:
        wr.wait()


def kernel(cache, update, pos):
    B, H, S, D = cache.shape
    U = update.shape[-2]
    BH = B * H
    cache3 = cache.reshape(BH, S, D)
    update3 = update.reshape(BH, U, D)
    pos_arr = jnp.asarray(pos, jnp.int32).reshape(1)

    out = pl.pallas_call(
        functools.partial(_body, S, U, BH),
        out_shape=jax.ShapeDtypeStruct((BH, S, D), cache.dtype),
        in_specs=[
            pl.BlockSpec(memory_space=pltpu.SMEM),
            pl.BlockSpec(memory_space=pl.ANY),
            pl.BlockSpec(memory_space=pl.ANY),
        ],
        out_specs=pl.BlockSpec(memory_space=pl.ANY),
        scratch_shapes=[
            pltpu.SemaphoreType.DMA,
            pltpu.SemaphoreType.DMA,
        ],
        name="kvcache_dma_copy_update",
    )(pos_arr, cache3, update3)
    return out.reshape(B, H, S, D)


# speedup vs baseline: 48.9087x; 48.9087x over previous
import functools

import jax
import jax.numpy as jnp
from jax import lax
from jax.experimental import pallas as pl
from jax.experimental.pallas import tpu as pltpu

_BHB = 4


def _body(S, U, pos_ref, cache_ref, upd_ref, out_ref):
    out_ref[...] = cache_ref[...]
    p = pos_ref[0]
    for i in range(U):
        r = lax.rem(p + i, S)
        out_ref[:, pl.ds(r, 1), :] = upd_ref[:, pl.ds(i, 1), :]


def kernel(cache, update, pos):
    B, H, S, D = cache.shape
    U = update.shape[-2]
    BH = B * H
    cache3 = cache.reshape(BH, S, D)
    update3 = update.reshape(BH, U, D)
    pos_arr = jnp.asarray(pos, jnp.int32).reshape(1)

    out = pl.pallas_call(
        functools.partial(_body, S, U),
        out_shape=jax.ShapeDtypeStruct((BH, S, D), cache.dtype),
        grid=(BH // _BHB,),
        in_specs=[
            pl.BlockSpec(memory_space=pltpu.SMEM),
            pl.BlockSpec((_BHB, S, D), lambda i: (i, 0, 0)),
            pl.BlockSpec((_BHB, U, D), lambda i: (i, 0, 0)),
        ],
        out_specs=pl.BlockSpec((_BHB, S, D), lambda i: (i, 0, 0)),
        compiler_params=pltpu.CompilerParams(
            dimension_semantics=("arbitrary",),
            vmem_limit_bytes=100 * 1024 * 1024,
        ),
        name="kvcache_fused_copy_update",
    )(pos_arr, cache3, update3)
    return out.reshape(B, H, S, D)


# speedup vs baseline: 49.2538x; 1.0071x over previous
import functools

import jax
import jax.numpy as jnp
from jax import lax
from jax.experimental import pallas as pl
from jax.experimental.pallas import tpu as pltpu

_CBH = 4
_NBUF = 3


def _body(S, U, BH, pos_ref, cache_any, upd_any, out_any,
          in_bufs, out_bufs, upd_v, in_sems, out_sems, usem):
    nchunk = BH // _CBH
    upd_cp = pltpu.make_async_copy(upd_any, upd_v, usem)
    upd_cp.start()

    def in_copy(k):
        return pltpu.make_async_copy(
            cache_any.at[pl.ds(k * _CBH, _CBH)],
            in_bufs.at[k % _NBUF],
            in_sems.at[k % _NBUF],
        )

    def out_copy(k):
        return pltpu.make_async_copy(
            out_bufs.at[k % _NBUF],
            out_any.at[pl.ds(k * _CBH, _CBH)],
            out_sems.at[k % _NBUF],
        )

    for k in range(_NBUF):
        in_copy(k).start()
    upd_cp.wait()
    p = pos_ref[0]

    for k in range(nchunk):
        b = k % _NBUF
        if k >= _NBUF:
            out_copy(k - _NBUF).wait()
        in_copy(k).wait()
        out_bufs[b] = in_bufs[b]
        for j in range(_CBH):
            bh = k * _CBH + j
            for i in range(U):
                r = lax.rem(p + i, S)
                out_bufs[b, j, pl.ds(r, 1), :] = upd_v[bh, pl.ds(i, 1), :]
        out_copy(k).start()
        if k + _NBUF < nchunk:
            in_copy(k + _NBUF).start()
    for k in range(nchunk - _NBUF, nchunk):
        out_copy(k).wait()


def kernel(cache, update, pos):
    B, H, S, D = cache.shape
    U = update.shape[-2]
    BH = B * H
    cache3 = cache.reshape(BH, S, D)
    update3 = update.reshape(BH, U, D)
    pos_arr = jnp.asarray(pos, jnp.int32).reshape(1)

    out = pl.pallas_call(
        functools.partial(_body, S, U, BH),
        out_shape=jax.ShapeDtypeStruct((BH, S, D), cache.dtype),
        in_specs=[
            pl.BlockSpec(memory_space=pltpu.SMEM),
            pl.BlockSpec(memory_space=pl.ANY),
            pl.BlockSpec(memory_space=pl.ANY),
        ],
        out_specs=pl.BlockSpec(memory_space=pl.ANY),
        scratch_shapes=[
            pltpu.VMEM((_NBUF, _CBH, S, D), jnp.float32),
            pltpu.VMEM((_NBUF, _CBH, S, D), jnp.float32),
            pltpu.VMEM((BH, U, D), jnp.float32),
            pltpu.SemaphoreType.DMA((_NBUF,)),
            pltpu.SemaphoreType.DMA((_NBUF,)),
            pltpu.SemaphoreType.DMA,
        ],
        compiler_params=pltpu.CompilerParams(
            vmem_limit_bytes=64 * 1024 * 1024,
        ),
        name="kvcache_ring_copy_update",
    )(pos_arr, cache3, update3)
    return out.reshape(B, H, S, D)
